# baseline (speedup 1.0000x reference)
import functools

import jax
import jax.numpy as jnp
from jax import lax
from jax.experimental import pallas as pl
from jax.experimental.pallas import tpu as pltpu
from jax.experimental.pallas import tpu_sc as plsc

_NUM_CORES = 2
_NUM_SUBCORES = 16
_NUM_WORKERS = _NUM_CORES * _NUM_SUBCORES

_NBUF = 4
_AHEAD = 3


def _make_gather(batch, seq, hidden):
  assert batch % _NUM_WORKERS == 0
  per_worker = batch // _NUM_WORKERS
  chunks = per_worker
  assert chunks % _NBUF == 0 and chunks >= _NBUF
  groups = chunks // _NBUF

  mesh = plsc.VectorSubcoreMesh(
      core_axis_name="c", subcore_axis_name="s",
      num_cores=_NUM_CORES, num_subcores=_NUM_SUBCORES)

  @functools.partial(
      pl.kernel,
      out_type=jax.ShapeDtypeStruct((batch, seq, hidden), jnp.float32),
      mesh=mesh,
      scratch_types=[
          pltpu.VMEM((per_worker, seq), jnp.int32),
          pltpu.VMEM((_NBUF, seq, hidden), jnp.float32),
          [pltpu.SemaphoreType.DMA] * _NBUF,
          [pltpu.SemaphoreType.DMA] * _NBUF,
      ],
      compiler_params=pltpu.CompilerParams(use_tc_tiling_on_sc=False),
  )
  def gather_kernel(table_hbm, idx_hbm, out_hbm, idx_v, rows_v, gsems, osems):
    wid = lax.axis_index("s") * _NUM_CORES + lax.axis_index("c")
    base = wid * per_worker

    pltpu.sync_copy(idx_hbm.at[pl.ds(base, per_worker)], idx_v)

    def gather_start(j, buf):
      pltpu.async_copy(table_hbm.at[idx_v.at[j]], rows_v.at[buf], gsems[buf])

    def gather_wait(j, buf):
      pltpu.make_async_copy(table_hbm.at[idx_v.at[j]], rows_v.at[buf],
                            gsems[buf]).wait()

    def out_start(j, buf):
      pltpu.async_copy(rows_v.at[buf], out_hbm.at[base + j], osems[buf])

    def out_wait(buf):
      pltpu.make_async_copy(rows_v.at[buf], out_hbm.at[base], osems[buf]).wait()

    for k in range(_AHEAD):
      gather_start(k, k)

    def body(g, _):
      for b in range(_NBUF):
        j = g * _NBUF + b
        nb = (b + _AHEAD) % _NBUF

        @pl.when(j + _AHEAD < chunks)
        def _():
          @pl.when(j + _AHEAD >= _NBUF)
          def _():
            out_wait(nb)
          gather_start(j + _AHEAD, nb)

        gather_wait(j, b)
        out_start(j, b)
      return ()

    lax.fori_loop(0, groups, body, ())

    for c in range(chunks - _NBUF, chunks):
      out_wait(c % _NBUF)

  return gather_kernel


def kernel(inputs, embeddings):
  batch, seq = inputs.shape
  hidden = embeddings.shape[1]
  return _make_gather(batch, seq, hidden)(embeddings, inputs.astype(jnp.int32))
